# Initial kernel scaffold; baseline (speedup 1.0000x reference)
import jax
import jax.numpy as jnp
from jax.experimental import pallas as pl


def kernel(tensor, int_to_feat_matrix, extra_embeddings):
    raise NotImplementedError("write your pallas kernel here")


# speedup vs baseline: 4.0481x; 4.0481x over previous
import functools

import jax
import jax.numpy as jnp
from jax import lax
from jax.experimental import pallas as pl
from jax.experimental.pallas import tpu as pltpu
from jax.experimental.pallas import tpu_sc as plsc

_MAX_COUNT = 255
_D = 128
_NC = 2
_NS = 16
_NW = _NC * _NS


def _build_blended_table(table, extra):
    def body(t_ref, e_ref, o_ref):
        row = lax.broadcasted_iota(jnp.int32, (_MAX_COUNT + 1, _D), 0)
        m = (row >= _MAX_COUNT).astype(jnp.float32)
        o_ref[...] = (1.0 - m) * t_ref[...] + m * e_ref[...]

    return pl.pallas_call(
        body,
        out_shape=jax.ShapeDtypeStruct((_MAX_COUNT + 1, _D), jnp.float32),
    )(table, extra)


@functools.lru_cache(maxsize=None)
def _make_gather(nrows2d):
    rows_per_w = nrows2d // _NW
    K = 4
    steps = rows_per_w // K
    R = K * 128
    assert rows_per_w % K == 0

    mesh = plsc.VectorSubcoreMesh(core_axis_name="c", subcore_axis_name="s")

    @functools.partial(
        pl.kernel,
        mesh=mesh,
        out_type=jax.ShapeDtypeStruct((nrows2d * _D, _D), jnp.float32),
        scratch_types=[
            pltpu.VMEM((K, 128), jnp.int32),
            pltpu.VMEM((R, _D), jnp.float32),
            pltpu.SemaphoreType.DMA,
        ],
    )
    def gather(idx_hbm, tbl_hbm, out_hbm, idx_v, rows_v, sem):
        wid = lax.axis_index("s") * _NC + lax.axis_index("c")
        row0 = wid * rows_per_w

        def step(s, carry):
            r = row0 + s * K
            pltpu.sync_copy(idx_hbm.at[pl.ds(r, K)], idx_v)
            handles = []
            for j in range(K):
                handles.append(
                    pltpu.async_copy(
                        tbl_hbm.at[idx_v.at[j]],
                        rows_v.at[pl.ds(j * 128, 128)],
                        sem,
                    )
                )
            for h in handles:
                h.wait()
            pltpu.sync_copy(rows_v, out_hbm.at[pl.ds(r * 128, R)])
            return carry

        lax.fori_loop(0, steps, step, 0)

    return gather


def kernel(tensor, int_to_feat_matrix, extra_embeddings):
    batch, fields = tensor.shape
    total = batch * fields
    nrows2d = total // 128
    assert total % 128 == 0

    blended = _build_blended_table(int_to_feat_matrix, extra_embeddings)
    idx2d = tensor.astype(jnp.int32).reshape(nrows2d, 128)
    out2d = _make_gather(nrows2d)(idx2d, blended)
    return out2d.reshape(batch, fields * _D)
